# baseline (speedup 1.0000x reference)
import functools

import jax
import jax.numpy as jnp
from jax.experimental import pallas as pl
from jax.experimental.pallas import tpu as pltpu

_BLK = 4096
_VPAD = 16


def _tc_body(idx_ref, tab_ref, out_ref):
    idx = idx_ref[0]
    oh_t = (jax.lax.broadcasted_iota(jnp.int32, (_VPAD, _BLK), 0) == idx)
    out_ref[...] = jax.lax.dot_general(
        oh_t.astype(jnp.float32), tab_ref[...],
        (((0,), (0,)), ((), ())),
        preferred_element_type=jnp.float32)


@functools.partial(jax.jit, static_argnames=("n", "d"))
def _tc_gather(idx3d, table_pad, n, d):
    grid = n // _BLK
    return pl.pallas_call(
        _tc_body,
        grid=(grid,),
        in_specs=[
            pl.BlockSpec((1, 1, _BLK), lambda i: (i, 0, 0)),
            pl.BlockSpec((_VPAD, d), lambda i: (0, 0)),
        ],
        out_specs=pl.BlockSpec((_BLK, d), lambda i: (i, 0)),
        out_shape=jax.ShapeDtypeStruct((n, d), jnp.float32),
        compiler_params=pltpu.CompilerParams(
            dimension_semantics=("arbitrary",),
        ),
    )(idx3d, table_pad)


def kernel(actions, embed_weight):
    b, a, l = actions.shape
    v, d = embed_weight.shape
    n = b * a * l
    idx3d = actions.reshape(n // _BLK, 1, _BLK)
    table_pad = jnp.pad(embed_weight, ((0, _VPAD - v), (0, 0)))
    out = _tc_gather(idx3d, table_pad, n, d)
    return out.reshape(b, a * l, d)
